# Initial kernel scaffold; baseline (speedup 1.0000x reference)
import jax
import jax.numpy as jnp
from jax.experimental import pallas as pl


def kernel(x, edge_attr, w0, w_rest, att_src, att_dst, bias, gamma, beta, edge_index, batch):
    raise NotImplementedError("write your pallas kernel here")


# speedup vs baseline: 33.5449x; 33.5449x over previous
import functools

import jax
import jax.numpy as jnp
from jax import lax
from jax.experimental import pallas as pl
from jax.experimental.pallas import tpu as pltpu
from jax.experimental.pallas import tpu_sc as plsc

N = 10000
E = 320000
H = 8
C = 16
D = 128
L = 5
G = 256

NC = 2
NS = 16
LANES = 16

NPAD = 10240
W = 128
NWIN = 162
ET = NWIN * W
EPAD = ET * NS
PADE = EPAD - E - N
PAD_ROWS = NPAD - N

RPT = NPAD // NS

_f32 = jnp.float32


def _xform_body(h_ref, stats_ref, gb_ref, w_ref, s_ref, hp_ref, al_ref):
    h = h_ref[...]
    st = stats_ref[...]
    mu = st[0:1] / N
    var = st[1:2] / N - mu * mu
    act = gb_ref[0:1] * (h - mu) * lax.rsqrt(var + 1e-5) + gb_ref[1:2]
    act = jnp.maximum(act, 0.0)
    hp = jnp.dot(act, w_ref[...], preferred_element_type=_f32)
    hp_ref[...] = hp
    al_ref[...] = jnp.dot(hp, s_ref[...], preferred_element_type=_f32)


def _xform1_body(x_ref, w_ref, s_ref, hp_ref, al_ref):
    hp = jnp.dot(x_ref[...], w_ref[...], preferred_element_type=_f32)
    hp_ref[...] = hp
    al_ref[...] = jnp.dot(hp, s_ref[...], preferred_element_type=_f32)


def _combine_body(seg_ref, hprev_ref, bias_ref, h_ref, stats_ref, acc):
    i = pl.program_id(0)
    h = (jnp.concatenate([seg_ref[0], seg_ref[1]], axis=1)
         + bias_ref[...] + hprev_ref[...])
    h_ref[...] = h

    @pl.when(i == 0)
    def _():
        acc[...] = jnp.zeros_like(acc)

    acc[0:1, :] += jnp.sum(h, axis=0, keepdims=True)
    acc[1:2, :] += jnp.sum(h * h, axis=0, keepdims=True)

    @pl.when(i == pl.num_programs(0) - 1)
    def _():
        stats_ref[...] = acc[...]


def _pool_body(h_ref, stats_ref, gb_ref, batch_ref, out_ref):
    i = pl.program_id(0)
    st = stats_ref[...]
    mu = st[0:1] / N
    var = st[1:2] / N - mu * mu
    hb = gb_ref[0:1] * (h_ref[...] - mu) * lax.rsqrt(var + 1e-5) + gb_ref[1:2]
    bt = batch_ref[0]
    gid = lax.broadcasted_iota(jnp.int32, (G, bt.shape[1]), 0)
    oh = (bt == gid).astype(_f32)
    contrib = lax.dot_general(oh, hb, (((1,), (0,)), ((), ())),
                              preferred_element_type=_f32)

    @pl.when(i == 0)
    def _():
        out_ref[...] = jnp.zeros_like(out_ref)

    out_ref[...] += contrib


_RB = 1000


def _xform(h, stats, gb, w, scat):
    nb = N // _RB
    return pl.pallas_call(
        _xform_body,
        grid=(nb,),
        in_specs=[
            pl.BlockSpec((_RB, D), lambda i: (i, 0)),
            pl.BlockSpec((2, D), lambda i: (0, 0)),
            pl.BlockSpec((2, D), lambda i: (0, 0)),
            pl.BlockSpec((D, D), lambda i: (0, 0)),
            pl.BlockSpec((D, 2 * H), lambda i: (0, 0)),
        ],
        out_specs=[
            pl.BlockSpec((_RB, D), lambda i: (i, 0)),
            pl.BlockSpec((_RB, 2 * H), lambda i: (i, 0)),
        ],
        out_shape=[
            jax.ShapeDtypeStruct((N, D), _f32),
            jax.ShapeDtypeStruct((N, 2 * H), _f32),
        ],
    )(h, stats, gb, w, scat)


def _xform1(x, w, scat):
    nb = N // _RB
    return pl.pallas_call(
        _xform1_body,
        grid=(nb,),
        in_specs=[
            pl.BlockSpec((_RB, 8), lambda i: (i, 0)),
            pl.BlockSpec((8, D), lambda i: (0, 0)),
            pl.BlockSpec((D, 2 * H), lambda i: (0, 0)),
        ],
        out_specs=[
            pl.BlockSpec((_RB, D), lambda i: (i, 0)),
            pl.BlockSpec((_RB, 2 * H), lambda i: (i, 0)),
        ],
        out_shape=[
            jax.ShapeDtypeStruct((N, D), _f32),
            jax.ShapeDtypeStruct((N, 2 * H), _f32),
        ],
    )(x, w, scat)


def _combine(seg, hprev, bias):
    nb = N // _RB
    return pl.pallas_call(
        _combine_body,
        grid=(nb,),
        in_specs=[
            pl.BlockSpec((2, _RB, 64), lambda i: (0, i, 0)),
            pl.BlockSpec((_RB, D), lambda i: (i, 0)),
            pl.BlockSpec((1, D), lambda i: (0, 0)),
        ],
        out_specs=[
            pl.BlockSpec((_RB, D), lambda i: (i, 0)),
            pl.BlockSpec((2, D), lambda i: (0, 0)),
        ],
        out_shape=[
            jax.ShapeDtypeStruct((N, D), _f32),
            jax.ShapeDtypeStruct((2, D), _f32),
        ],
        scratch_shapes=[pltpu.VMEM((2, D), _f32)],
    )(seg, hprev, bias)


def _pool(h, stats, gb, batch3):
    nb = N // _RB
    return pl.pallas_call(
        _pool_body,
        grid=(nb,),
        in_specs=[
            pl.BlockSpec((_RB, D), lambda i: (i, 0)),
            pl.BlockSpec((2, D), lambda i: (0, 0)),
            pl.BlockSpec((2, D), lambda i: (0, 0)),
            pl.BlockSpec((1, 1, _RB), lambda i: (i, 0, 0)),
        ],
        out_specs=pl.BlockSpec((G, D), lambda i: (0, 0)),
        out_shape=jax.ShapeDtypeStruct((G, D), _f32),
    )(h, stats, gb, batch3)


def _bcast_lane(v, h):
    idx = lax.full((LANES, 1), h, jnp.int32)
    dn = lax.GatherDimensionNumbers(
        offset_dims=(), collapsed_slice_dims=(0,), start_index_map=(0,))
    return lax.gather(v, idx, dn, (1,),
                      mode=lax.GatherScatterMode.PROMISE_IN_BOUNDS)


def _sc_body(hp_hbm, als_hbm, ald_hbm, src_hbm, dst_hbm, z16_hbm, z64_hbm,
             seg_hbm, exh_hbm,
             hp_t, out_t, als_t, ald_t, den_t,
             sidx, didx, rs, rd, hg, sem_a, sem_b):
    c = lax.axis_index("c")
    s = lax.axis_index("s")
    r0 = s * RPT
    tbase = s * ET

    def stage16(hview, tview):
        def body(k, _):
            pltpu.sync_copy(hview.at[pl.ds(r0 + k * W, W)], rs)
            pltpu.sync_copy(rs, tview.at[pl.ds(r0 + k * W, W)])
            return 0
        lax.fori_loop(0, RPT // W, body, 0)

    def stage64(hview, tview):
        def body(k, _):
            pltpu.sync_copy(hview.at[pl.ds(r0 + k * W, W)], hg)
            pltpu.sync_copy(hg, tview.at[pl.ds(r0 + k * W, W)])
            return 0
        lax.fori_loop(0, RPT // W, body, 0)

    stage64(hp_hbm.at[c], hp_t)
    stage16(als_hbm.at[c], als_t)
    stage16(ald_hbm.at[c], ald_t)
    stage16(z16_hbm, den_t)
    stage64(z64_hbm, out_t)
    plsc.subcore_barrier()

    def awin(w, _):
        eb = tbase + w * W
        pltpu.sync_copy(src_hbm.at[pl.ds(eb, W)], sidx.at[0])
        pltpu.sync_copy(dst_hbm.at[pl.ds(eb, W)], didx.at[0])
        ca = pltpu.async_copy(als_t.at[sidx.at[0]], rs, sem_a)
        cb = pltpu.async_copy(ald_t.at[didx.at[0]], rd, sem_b)
        ca.wait()
        cb.wait()

        def ebody(j, _):
            v = rs[j] + rd[j]
            v = jnp.where(v > 0.0, v, 0.2 * v)
            rs[j] = jnp.exp(v)
            return 0
        lax.fori_loop(0, W, ebody, 0)
        pltpu.sync_copy(rs, den_t.at[didx.at[0]], add=True)
        pltpu.sync_copy(rs, exh_hbm.at[c, pl.ds(eb, W)])
        return 0
    lax.fori_loop(0, NWIN, awin, 0)
    plsc.subcore_barrier()

    def dchunk(k, _):
        pltpu.sync_copy(den_t.at[pl.ds(r0 + k * W, W)], rs)

        def dbody(j, _):
            rs[j] = 1.0 / (rs[j] + 1e-16)
            return 0
        lax.fori_loop(0, W, dbody, 0)
        pltpu.sync_copy(rs, den_t.at[pl.ds(r0 + k * W, W)])
        return 0
    lax.fori_loop(0, RPT // W, dchunk, 0)
    plsc.subcore_barrier()

    def bwin(w, _):
        eb = tbase + w * W
        pltpu.sync_copy(src_hbm.at[pl.ds(eb, W)], sidx.at[0])
        pltpu.sync_copy(dst_hbm.at[pl.ds(eb, W)], didx.at[0])
        pltpu.sync_copy(exh_hbm.at[c, pl.ds(eb, W)], rs)
        ca = pltpu.async_copy(den_t.at[didx.at[0]], rd, sem_a)
        cb = pltpu.async_copy(hp_t.at[sidx.at[0]], hg, sem_b)
        ca.wait()
        cb.wait()

        def ebody(j, _):
            alpha = rs[j] * rd[j]
            for h in range(4):
                ah = _bcast_lane(alpha, h)
                hg[j, pl.ds(h * LANES, LANES)] = (
                    hg[j, pl.ds(h * LANES, LANES)] * ah)
            return 0
        lax.fori_loop(0, W, ebody, 0)
        pltpu.sync_copy(hg, out_t.at[didx.at[0]], add=True)
        return 0
    lax.fori_loop(0, NWIN, bwin, 0)
    plsc.subcore_barrier()

    def wb(k, _):
        pltpu.sync_copy(out_t.at[pl.ds(r0 + k * W, W)], hg)
        pltpu.sync_copy(hg, seg_hbm.at[c, pl.ds(r0 + k * W, W)])
        return 0
    lax.fori_loop(0, RPT // W, wb, 0)


@functools.partial(jax.jit, static_argnames=())
def _sc_layer(hp2, als16, ald16, src, dst, z16, z64):
    mesh = plsc.VectorSubcoreMesh(core_axis_name="c", subcore_axis_name="s",
                                  num_cores=NC, num_subcores=NS)
    seg, _ex = pl.kernel(
        _sc_body,
        out_type=[
            jax.ShapeDtypeStruct((NC, NPAD, 64), _f32),
            jax.ShapeDtypeStruct((NC, EPAD, 16), _f32),
        ],
        mesh=mesh,
        scratch_types=[
            pltpu.VMEM_SHARED((NPAD, 64), _f32),
            pltpu.VMEM_SHARED((NPAD, 64), _f32),
            pltpu.VMEM_SHARED((NPAD, 16), _f32),
            pltpu.VMEM_SHARED((NPAD, 16), _f32),
            pltpu.VMEM_SHARED((NPAD, 16), _f32),
            pltpu.VMEM((1, W), jnp.int32),
            pltpu.VMEM((1, W), jnp.int32),
            pltpu.VMEM((W, 16), _f32),
            pltpu.VMEM((W, 16), _f32),
            pltpu.VMEM((W, 64), _f32),
            pltpu.SemaphoreType.DMA,
            pltpu.SemaphoreType.DMA,
        ],
        compiler_params=pltpu.CompilerParams(use_tc_tiling_on_sc=False),
    )(hp2, als16, ald16, src, dst, z16, z64)
    return seg


def _mk_scat(att_s, att_d):
    eye = jnp.eye(H, dtype=_f32)
    ss = (att_s[:, :, None] * eye[:, None, :]).reshape(D, H)
    sd = (att_d[:, :, None] * eye[:, None, :]).reshape(D, H)
    return jnp.concatenate([ss, sd], axis=1)


def _split_tables(hp, al):
    hp2 = jnp.pad(hp.reshape(N, 2, 64).transpose(1, 0, 2),
                  ((0, 0), (0, NPAD - N), (0, 0)))
    als = jnp.pad(al[:, :H].reshape(N, 2, 4).transpose(1, 0, 2),
                  ((0, 0), (0, NPAD - N), (0, 12)))
    ald = jnp.pad(al[:, H:].reshape(N, 2, 4).transpose(1, 0, 2),
                  ((0, 0), (0, NPAD - N), (0, 12)))
    return hp2, als, ald


def kernel(x, edge_attr, w0, w_rest, att_src, att_dst, bias, gamma, beta,
           edge_index, batch):
    loops = jnp.arange(N, dtype=jnp.int32)
    padi = jnp.arange(PADE, dtype=jnp.int32)
    src = jnp.concatenate([edge_index[0], loops, padi % N])
    dst = jnp.concatenate([edge_index[1], loops, N + padi % PAD_ROWS])

    z16 = jnp.zeros((NPAD, 16), _f32)
    z64 = jnp.zeros((NPAD, 64), _f32)
    batch3 = batch.reshape(N // _RB, 1, _RB)

    hp, al = _xform1(x, w0, _mk_scat(att_src[0], att_dst[0]))
    hp2, als16, ald16 = _split_tables(hp, al)
    seg = _sc_layer(hp2, als16, ald16, src, dst, z16, z64)
    h, stats = _combine(seg, jnp.zeros((N, D), _f32), bias[0:1])

    for l in range(1, L):
        gb = jnp.stack([gamma[l - 1], beta[l - 1]])
        hp, al = _xform(h, stats, gb, w_rest[l - 1],
                        _mk_scat(att_src[l], att_dst[l]))
        hp2, als16, ald16 = _split_tables(hp, al)
        seg = _sc_layer(hp2, als16, ald16, src, dst, z16, z64)
        h, stats = _combine(seg, h, bias[l:l + 1])

    gb = jnp.stack([gamma[L - 1], beta[L - 1]])
    return _pool(h, stats, gb, batch3)


# speedup vs baseline: 41.1119x; 1.2256x over previous
import functools

import jax
import jax.numpy as jnp
from jax import lax
from jax.experimental import pallas as pl
from jax.experimental.pallas import tpu as pltpu
from jax.experimental.pallas import tpu_sc as plsc

N = 10000
E = 320000
H = 8
C = 16
D = 128
L = 5
G = 256

NC = 2
NS = 16
LANES = 16

NPAD = 10240
W = 128
NWIN = 162
ET = NWIN * W
EPAD = ET * NS
PADE = EPAD - E - N
PAD_ROWS = NPAD - N

RPT = NPAD // NS

_f32 = jnp.float32


def _xform_body(h_ref, stats_ref, gb_ref, w_ref, s_ref, hp_ref, al_ref):
    h = h_ref[...]
    st = stats_ref[...]
    mu = st[0:1] / N
    var = st[1:2] / N - mu * mu
    act = gb_ref[0:1] * (h - mu) * lax.rsqrt(var + 1e-5) + gb_ref[1:2]
    act = jnp.maximum(act, 0.0)
    hp = jnp.dot(act, w_ref[...], preferred_element_type=_f32)
    hp_ref[...] = hp
    al_ref[...] = jnp.dot(hp, s_ref[...], preferred_element_type=_f32)


def _xform1_body(x_ref, w_ref, s_ref, hp_ref, al_ref):
    hp = jnp.dot(x_ref[...], w_ref[...], preferred_element_type=_f32)
    hp_ref[...] = hp
    al_ref[...] = jnp.dot(hp, s_ref[...], preferred_element_type=_f32)


def _combine_body(seg_ref, hprev_ref, bias_ref, h_ref, stats_ref, acc):
    i = pl.program_id(0)
    h = (jnp.concatenate([seg_ref[0], seg_ref[1]], axis=1)
         + bias_ref[...] + hprev_ref[...])
    h_ref[...] = h

    @pl.when(i == 0)
    def _():
        acc[...] = jnp.zeros_like(acc)

    acc[0:1, :] += jnp.sum(h, axis=0, keepdims=True)
    acc[1:2, :] += jnp.sum(h * h, axis=0, keepdims=True)

    @pl.when(i == pl.num_programs(0) - 1)
    def _():
        stats_ref[...] = acc[...]


def _pool_body(h_ref, stats_ref, gb_ref, batch_ref, out_ref):
    i = pl.program_id(0)
    st = stats_ref[...]
    mu = st[0:1] / N
    var = st[1:2] / N - mu * mu
    hb = gb_ref[0:1] * (h_ref[...] - mu) * lax.rsqrt(var + 1e-5) + gb_ref[1:2]
    bt = batch_ref[0]
    gid = lax.broadcasted_iota(jnp.int32, (G, bt.shape[1]), 0)
    oh = (bt == gid).astype(_f32)
    contrib = lax.dot_general(oh, hb, (((1,), (0,)), ((), ())),
                              preferred_element_type=_f32)

    @pl.when(i == 0)
    def _():
        out_ref[...] = jnp.zeros_like(out_ref)

    out_ref[...] += contrib


_RB = 1000


def _xform(h, stats, gb, w, scat):
    nb = N // _RB
    return pl.pallas_call(
        _xform_body,
        grid=(nb,),
        in_specs=[
            pl.BlockSpec((_RB, D), lambda i: (i, 0)),
            pl.BlockSpec((2, D), lambda i: (0, 0)),
            pl.BlockSpec((2, D), lambda i: (0, 0)),
            pl.BlockSpec((D, D), lambda i: (0, 0)),
            pl.BlockSpec((D, 2 * H), lambda i: (0, 0)),
        ],
        out_specs=[
            pl.BlockSpec((_RB, D), lambda i: (i, 0)),
            pl.BlockSpec((_RB, 2 * H), lambda i: (i, 0)),
        ],
        out_shape=[
            jax.ShapeDtypeStruct((N, D), _f32),
            jax.ShapeDtypeStruct((N, 2 * H), _f32),
        ],
    )(h, stats, gb, w, scat)


def _xform1(x, w, scat):
    nb = N // _RB
    return pl.pallas_call(
        _xform1_body,
        grid=(nb,),
        in_specs=[
            pl.BlockSpec((_RB, 8), lambda i: (i, 0)),
            pl.BlockSpec((8, D), lambda i: (0, 0)),
            pl.BlockSpec((D, 2 * H), lambda i: (0, 0)),
        ],
        out_specs=[
            pl.BlockSpec((_RB, D), lambda i: (i, 0)),
            pl.BlockSpec((_RB, 2 * H), lambda i: (i, 0)),
        ],
        out_shape=[
            jax.ShapeDtypeStruct((N, D), _f32),
            jax.ShapeDtypeStruct((N, 2 * H), _f32),
        ],
    )(x, w, scat)


def _combine(seg, hprev, bias):
    nb = N // _RB
    return pl.pallas_call(
        _combine_body,
        grid=(nb,),
        in_specs=[
            pl.BlockSpec((2, _RB, 64), lambda i: (0, i, 0)),
            pl.BlockSpec((_RB, D), lambda i: (i, 0)),
            pl.BlockSpec((1, D), lambda i: (0, 0)),
        ],
        out_specs=[
            pl.BlockSpec((_RB, D), lambda i: (i, 0)),
            pl.BlockSpec((2, D), lambda i: (0, 0)),
        ],
        out_shape=[
            jax.ShapeDtypeStruct((N, D), _f32),
            jax.ShapeDtypeStruct((2, D), _f32),
        ],
        scratch_shapes=[pltpu.VMEM((2, D), _f32)],
    )(seg, hprev, bias)


def _pool(h, stats, gb, batch3):
    nb = N // _RB
    return pl.pallas_call(
        _pool_body,
        grid=(nb,),
        in_specs=[
            pl.BlockSpec((_RB, D), lambda i: (i, 0)),
            pl.BlockSpec((2, D), lambda i: (0, 0)),
            pl.BlockSpec((2, D), lambda i: (0, 0)),
            pl.BlockSpec((1, 1, _RB), lambda i: (i, 0, 0)),
        ],
        out_specs=pl.BlockSpec((G, D), lambda i: (0, 0)),
        out_shape=jax.ShapeDtypeStruct((G, D), _f32),
    )(h, stats, gb, batch3)


def _bcast_lane(v, h):
    idx = lax.full((LANES, 1), h, jnp.int32)
    dn = lax.GatherDimensionNumbers(
        offset_dims=(), collapsed_slice_dims=(0,), start_index_map=(0,))
    return lax.gather(v, idx, dn, (1,),
                      mode=lax.GatherScatterMode.PROMISE_IN_BOUNDS)


def _sc_body(hp_hbm, als_hbm, ald_hbm, src_hbm, dst_hbm, z16_hbm, z64_hbm,
             seg_hbm, exh_hbm,
             hp_t, out_t, als_t, ald_t, den_t,
             sidx, didx, rs, rd, hg, sem_a, sem_b):
    c = lax.axis_index("c")
    s = lax.axis_index("s")
    r0 = s * RPT
    tbase = s * ET

    def stage16(hview, tview):
        def body(k, _):
            pltpu.sync_copy(hview.at[pl.ds(r0 + k * W, W)], rs)
            pltpu.sync_copy(rs, tview.at[pl.ds(r0 + k * W, W)])
            return 0
        lax.fori_loop(0, RPT // W, body, 0)

    def stage64(hview, tview):
        def body(k, _):
            pltpu.sync_copy(hview.at[pl.ds(r0 + k * W, W)], hg)
            pltpu.sync_copy(hg, tview.at[pl.ds(r0 + k * W, W)])
            return 0
        lax.fori_loop(0, RPT // W, body, 0)

    stage64(hp_hbm.at[c], hp_t)
    stage16(als_hbm.at[c], als_t)
    stage16(ald_hbm.at[c], ald_t)
    stage16(z16_hbm, den_t)
    stage64(z64_hbm, out_t)
    plsc.subcore_barrier()

    def awin(w, _):
        eb = tbase + w * W
        pltpu.sync_copy(src_hbm.at[pl.ds(eb, W)], sidx.at[0])
        pltpu.sync_copy(dst_hbm.at[pl.ds(eb, W)], didx.at[0])
        ca = pltpu.async_copy(als_t.at[sidx.at[0]], rs, sem_a)
        cb = pltpu.async_copy(ald_t.at[didx.at[0]], rd, sem_b)
        ca.wait()
        cb.wait()

        def ebody(j, _):
            for u in range(8):
                e = j * 8 + u
                v = rs[e] + rd[e]
                v = jnp.maximum(v, 0.2 * v)
                rs[e] = jnp.exp(v)
            return 0
        lax.fori_loop(0, W // 8, ebody, 0)
        pltpu.sync_copy(rs, den_t.at[didx.at[0]], add=True)
        pltpu.sync_copy(rs, exh_hbm.at[c, pl.ds(eb, W)])
        return 0
    lax.fori_loop(0, NWIN, awin, 0)
    plsc.subcore_barrier()

    def dchunk(k, _):
        pltpu.sync_copy(den_t.at[pl.ds(r0 + k * W, W)], rs)

        def dbody(j, _):
            rs[j] = 1.0 / (rs[j] + 1e-16)
            return 0
        lax.fori_loop(0, W, dbody, 0)
        pltpu.sync_copy(rs, den_t.at[pl.ds(r0 + k * W, W)])
        return 0
    lax.fori_loop(0, RPT // W, dchunk, 0)
    plsc.subcore_barrier()

    def bwin(w, _):
        eb = tbase + w * W
        pltpu.sync_copy(src_hbm.at[pl.ds(eb, W)], sidx.at[0])
        pltpu.sync_copy(dst_hbm.at[pl.ds(eb, W)], didx.at[0])
        pltpu.sync_copy(exh_hbm.at[c, pl.ds(eb, W)], rs)
        ca = pltpu.async_copy(den_t.at[didx.at[0]], rd, sem_a)
        cb = pltpu.async_copy(hp_t.at[sidx.at[0]], hg, sem_b)
        ca.wait()
        cb.wait()

        def ebody(j, _):
            for u in range(2):
                e = j * 2 + u
                alpha = rs[e] * rd[e]
                for h in range(4):
                    ah = _bcast_lane(alpha, h)
                    hg[e, pl.ds(h * LANES, LANES)] = (
                        hg[e, pl.ds(h * LANES, LANES)] * ah)
            return 0
        lax.fori_loop(0, W // 2, ebody, 0)
        pltpu.sync_copy(hg, out_t.at[didx.at[0]], add=True)
        return 0
    lax.fori_loop(0, NWIN, bwin, 0)
    plsc.subcore_barrier()

    def wb(k, _):
        pltpu.sync_copy(out_t.at[pl.ds(r0 + k * W, W)], hg)
        pltpu.sync_copy(hg, seg_hbm.at[c, pl.ds(r0 + k * W, W)])
        return 0
    lax.fori_loop(0, RPT // W, wb, 0)


@functools.partial(jax.jit, static_argnames=())
def _sc_layer(hp2, als16, ald16, src, dst, z16, z64):
    mesh = plsc.VectorSubcoreMesh(core_axis_name="c", subcore_axis_name="s",
                                  num_cores=NC, num_subcores=NS)
    seg, _ex = pl.kernel(
        _sc_body,
        out_type=[
            jax.ShapeDtypeStruct((NC, NPAD, 64), _f32),
            jax.ShapeDtypeStruct((NC, EPAD, 16), _f32),
        ],
        mesh=mesh,
        scratch_types=[
            pltpu.VMEM_SHARED((NPAD, 64), _f32),
            pltpu.VMEM_SHARED((NPAD, 64), _f32),
            pltpu.VMEM_SHARED((NPAD, 16), _f32),
            pltpu.VMEM_SHARED((NPAD, 16), _f32),
            pltpu.VMEM_SHARED((NPAD, 16), _f32),
            pltpu.VMEM((1, W), jnp.int32),
            pltpu.VMEM((1, W), jnp.int32),
            pltpu.VMEM((W, 16), _f32),
            pltpu.VMEM((W, 16), _f32),
            pltpu.VMEM((W, 64), _f32),
            pltpu.SemaphoreType.DMA,
            pltpu.SemaphoreType.DMA,
        ],
        compiler_params=pltpu.CompilerParams(use_tc_tiling_on_sc=False),
    )(hp2, als16, ald16, src, dst, z16, z64)
    return seg


def _mk_scat(att_s, att_d):
    eye = jnp.eye(H, dtype=_f32)
    ss = (att_s[:, :, None] * eye[:, None, :]).reshape(D, H)
    sd = (att_d[:, :, None] * eye[:, None, :]).reshape(D, H)
    return jnp.concatenate([ss, sd], axis=1)


def _split_tables(hp, al):
    hp2 = jnp.pad(hp.reshape(N, 2, 64).transpose(1, 0, 2),
                  ((0, 0), (0, NPAD - N), (0, 0)))
    als = jnp.pad(al[:, :H].reshape(N, 2, 4).transpose(1, 0, 2),
                  ((0, 0), (0, NPAD - N), (0, 12)))
    ald = jnp.pad(al[:, H:].reshape(N, 2, 4).transpose(1, 0, 2),
                  ((0, 0), (0, NPAD - N), (0, 12)))
    return hp2, als, ald


def kernel(x, edge_attr, w0, w_rest, att_src, att_dst, bias, gamma, beta,
           edge_index, batch):
    loops = jnp.arange(N, dtype=jnp.int32)
    padi = jnp.arange(PADE, dtype=jnp.int32)
    src = jnp.concatenate([edge_index[0], loops, padi % N])
    dst = jnp.concatenate([edge_index[1], loops, N + padi % PAD_ROWS])

    z16 = jnp.zeros((NPAD, 16), _f32)
    z64 = jnp.zeros((NPAD, 64), _f32)
    batch3 = batch.reshape(N // _RB, 1, _RB)

    hp, al = _xform1(x, w0, _mk_scat(att_src[0], att_dst[0]))
    hp2, als16, ald16 = _split_tables(hp, al)
    seg = _sc_layer(hp2, als16, ald16, src, dst, z16, z64)
    h, stats = _combine(seg, jnp.zeros((N, D), _f32), bias[0:1])

    for l in range(1, L):
        gb = jnp.stack([gamma[l - 1], beta[l - 1]])
        hp, al = _xform(h, stats, gb, w_rest[l - 1],
                        _mk_scat(att_src[l], att_dst[l]))
        hp2, als16, ald16 = _split_tables(hp, al)
        seg = _sc_layer(hp2, als16, ald16, src, dst, z16, z64)
        h, stats = _combine(seg, h, bias[l:l + 1])

    gb = jnp.stack([gamma[L - 1], beta[L - 1]])
    return _pool(h, stats, gb, batch3)


# speedup vs baseline: 65.8690x; 1.6022x over previous
import functools

import jax
import jax.numpy as jnp
from jax import lax
from jax.experimental import pallas as pl
from jax.experimental.pallas import tpu as pltpu
from jax.experimental.pallas import tpu_sc as plsc

N = 10000
E = 320000
H = 8
C = 16
D = 128
L = 5
G = 256

NC = 2
NS = 16
LANES = 16

NPAD = 10240
W = 128
NWIN = 162
ET = NWIN * W
EPAD = ET * NS
PADE = EPAD - E - N
PAD_ROWS = NPAD - N

RPT = NPAD // NS

_f32 = jnp.float32


def _xform_body(h_ref, stats_ref, gb_ref, w_ref, s_ref, hp_ref, al_ref):
    h = h_ref[...]
    st = stats_ref[...]
    mu = st[0:1] / N
    var = st[1:2] / N - mu * mu
    act = gb_ref[0:1] * (h - mu) * lax.rsqrt(var + 1e-5) + gb_ref[1:2]
    act = jnp.maximum(act, 0.0)
    hp = jnp.dot(act, w_ref[...], preferred_element_type=_f32)
    hp_ref[...] = hp
    al_ref[...] = jnp.dot(hp, s_ref[...], preferred_element_type=_f32)


def _xform1_body(x_ref, w_ref, s_ref, hp_ref, al_ref):
    hp = jnp.dot(x_ref[...], w_ref[...], preferred_element_type=_f32)
    hp_ref[...] = hp
    al_ref[...] = jnp.dot(hp, s_ref[...], preferred_element_type=_f32)


def _combine_body(seg_ref, hprev_ref, bias_ref, h_ref, stats_ref, acc):
    i = pl.program_id(0)
    h = (jnp.concatenate([seg_ref[0], seg_ref[1]], axis=1)
         + bias_ref[...] + hprev_ref[...])
    h_ref[...] = h

    @pl.when(i == 0)
    def _():
        acc[...] = jnp.zeros_like(acc)

    acc[0:1, :] += jnp.sum(h, axis=0, keepdims=True)
    acc[1:2, :] += jnp.sum(h * h, axis=0, keepdims=True)

    @pl.when(i == pl.num_programs(0) - 1)
    def _():
        stats_ref[...] = acc[...]


def _pool_body(h_ref, stats_ref, gb_ref, batch_ref, out_ref):
    i = pl.program_id(0)
    st = stats_ref[...]
    mu = st[0:1] / N
    var = st[1:2] / N - mu * mu
    hb = gb_ref[0:1] * (h_ref[...] - mu) * lax.rsqrt(var + 1e-5) + gb_ref[1:2]
    bt = batch_ref[0]
    gid = lax.broadcasted_iota(jnp.int32, (G, bt.shape[1]), 0)
    oh = (bt == gid).astype(_f32)
    contrib = lax.dot_general(oh, hb, (((1,), (0,)), ((), ())),
                              preferred_element_type=_f32)

    @pl.when(i == 0)
    def _():
        out_ref[...] = jnp.zeros_like(out_ref)

    out_ref[...] += contrib


_RB = 1000


def _xform(h, stats, gb, w, scat):
    nb = N // _RB
    return pl.pallas_call(
        _xform_body,
        grid=(nb,),
        in_specs=[
            pl.BlockSpec((_RB, D), lambda i: (i, 0)),
            pl.BlockSpec((2, D), lambda i: (0, 0)),
            pl.BlockSpec((2, D), lambda i: (0, 0)),
            pl.BlockSpec((D, D), lambda i: (0, 0)),
            pl.BlockSpec((D, 2 * H), lambda i: (0, 0)),
        ],
        out_specs=[
            pl.BlockSpec((_RB, D), lambda i: (i, 0)),
            pl.BlockSpec((_RB, 2 * H), lambda i: (i, 0)),
        ],
        out_shape=[
            jax.ShapeDtypeStruct((N, D), _f32),
            jax.ShapeDtypeStruct((N, 2 * H), _f32),
        ],
    )(h, stats, gb, w, scat)


def _xform1(x, w, scat):
    nb = N // _RB
    return pl.pallas_call(
        _xform1_body,
        grid=(nb,),
        in_specs=[
            pl.BlockSpec((_RB, 8), lambda i: (i, 0)),
            pl.BlockSpec((8, D), lambda i: (0, 0)),
            pl.BlockSpec((D, 2 * H), lambda i: (0, 0)),
        ],
        out_specs=[
            pl.BlockSpec((_RB, D), lambda i: (i, 0)),
            pl.BlockSpec((_RB, 2 * H), lambda i: (i, 0)),
        ],
        out_shape=[
            jax.ShapeDtypeStruct((N, D), _f32),
            jax.ShapeDtypeStruct((N, 2 * H), _f32),
        ],
    )(x, w, scat)


def _combine(seg, hprev, bias):
    nb = N // _RB
    return pl.pallas_call(
        _combine_body,
        grid=(nb,),
        in_specs=[
            pl.BlockSpec((2, _RB, 64), lambda i: (0, i, 0)),
            pl.BlockSpec((_RB, D), lambda i: (i, 0)),
            pl.BlockSpec((1, D), lambda i: (0, 0)),
        ],
        out_specs=[
            pl.BlockSpec((_RB, D), lambda i: (i, 0)),
            pl.BlockSpec((2, D), lambda i: (0, 0)),
        ],
        out_shape=[
            jax.ShapeDtypeStruct((N, D), _f32),
            jax.ShapeDtypeStruct((2, D), _f32),
        ],
        scratch_shapes=[pltpu.VMEM((2, D), _f32)],
    )(seg, hprev, bias)


def _pool(h, stats, gb, batch3):
    nb = N // _RB
    return pl.pallas_call(
        _pool_body,
        grid=(nb,),
        in_specs=[
            pl.BlockSpec((_RB, D), lambda i: (i, 0)),
            pl.BlockSpec((2, D), lambda i: (0, 0)),
            pl.BlockSpec((2, D), lambda i: (0, 0)),
            pl.BlockSpec((1, 1, _RB), lambda i: (i, 0, 0)),
        ],
        out_specs=pl.BlockSpec((G, D), lambda i: (0, 0)),
        out_shape=jax.ShapeDtypeStruct((G, D), _f32),
    )(h, stats, gb, batch3)


def _bcast_lane(v, h):
    idx = lax.full((LANES, 1), h, jnp.int32)
    dn = lax.GatherDimensionNumbers(
        offset_dims=(), collapsed_slice_dims=(0,), start_index_map=(0,))
    return lax.gather(v, idx, dn, (1,),
                      mode=lax.GatherScatterMode.PROMISE_IN_BOUNDS)


def _sc_body(hp_hbm, als_hbm, ald_hbm, sdx_hbm, z16_hbm, z64_hbm,
             seg_hbm, exh_hbm,
             hp_t, out_t, als_t, ald_t, den_t,
             id0, id1, rs0, rs1, rd, hg, sem_a, sem_b, sp0, sp1):
    ids = (id0, id1)
    rss = (rs0, rs1)
    sps = (sp0, sp1)
    rs = rs0
    c = lax.axis_index("c")
    s = lax.axis_index("s")
    r0 = s * RPT
    tbase = s * ET

    def stage16(hview, tview):
        def body(k, _):
            pltpu.sync_copy(hview.at[pl.ds(r0 + k * W, W)], rs)
            pltpu.sync_copy(rs, tview.at[pl.ds(r0 + k * W, W)])
            return 0
        lax.fori_loop(0, RPT // W, body, 0)

    def stage64(hview, tview):
        def body(k, _):
            pltpu.sync_copy(hview.at[pl.ds(r0 + k * W, W)], hg)
            pltpu.sync_copy(hg, tview.at[pl.ds(r0 + k * W, W)])
            return 0
        lax.fori_loop(0, RPT // W, body, 0)

    stage64(hp_hbm.at[c], hp_t)
    stage16(als_hbm.at[c], als_t)
    stage16(ald_hbm.at[c], ald_t)
    stage16(z16_hbm, den_t)
    stage64(z64_hbm, out_t)
    plsc.subcore_barrier()

    wbase = s * NWIN

    def a_pref(w, b):
        pltpu.async_copy(sdx_hbm.at[wbase + w], ids[b], sps[b])

    def a_wait_pref(b):
        pltpu.make_async_copy(sdx_hbm.at[0], ids[b], sps[b]).wait()

    def a_step(w, b):
        a_pref(w + 1, 1 - b)
        a_wait_pref(b)
        idb = ids[b]
        rsb = rss[b]
        ca = pltpu.async_copy(als_t.at[idb.at[0]], rsb, sem_a)
        cb = pltpu.async_copy(ald_t.at[idb.at[1]], rd, sem_b)
        ca.wait()
        cb.wait()

        def ebody(j, _):
            for u in range(8):
                e = j * 8 + u
                v = rsb[e] + rd[e]
                v = jnp.maximum(v, 0.2 * v)
                rsb[e] = jnp.exp(v)
            return 0
        lax.fori_loop(0, W // 8, ebody, 0)
        eb = tbase + w * W
        pltpu.sync_copy(rsb, den_t.at[idb.at[1]], add=True)
        pltpu.sync_copy(rsb, exh_hbm.at[c, pl.ds(eb, W)])

    a_pref(0, 0)

    def a_steady(g, _):
        a_step(2 * g, 0)
        a_step(2 * g + 1, 1)
        return 0
    lax.fori_loop(0, NWIN // 2, a_steady, 0)
    a_wait_pref(0)
    plsc.subcore_barrier()

    def dchunk(k, _):
        pltpu.sync_copy(den_t.at[pl.ds(r0 + k * W, W)], rs)

        def dbody(j, _):
            rs[j] = 1.0 / (rs[j] + 1e-16)
            return 0
        lax.fori_loop(0, W, dbody, 0)
        pltpu.sync_copy(rs, den_t.at[pl.ds(r0 + k * W, W)])
        return 0
    lax.fori_loop(0, RPT // W, dchunk, 0)
    plsc.subcore_barrier()

    def b_pref(w, b):
        eb = tbase + w * W
        pltpu.async_copy(sdx_hbm.at[wbase + w], ids[b], sps[b])
        pltpu.async_copy(exh_hbm.at[c, pl.ds(eb, W)], rss[b], sps[b])

    def b_wait_pref(b):
        pltpu.make_async_copy(sdx_hbm.at[0], ids[b], sps[b]).wait()
        pltpu.make_async_copy(exh_hbm.at[c, pl.ds(0, W)], rss[b],
                              sps[b]).wait()

    def b_step(w, b):
        b_pref(w + 1, 1 - b)
        b_wait_pref(b)
        idb = ids[b]
        rsb = rss[b]
        ca = pltpu.async_copy(den_t.at[idb.at[1]], rd, sem_a)
        cb = pltpu.async_copy(hp_t.at[idb.at[0]], hg, sem_b)
        ca.wait()
        cb.wait()

        def ebody(j, _):
            for u in range(2):
                e = j * 2 + u
                alpha = rsb[e] * rd[e]
                for h in range(4):
                    ah = _bcast_lane(alpha, h)
                    hg[e, pl.ds(h * LANES, LANES)] = (
                        hg[e, pl.ds(h * LANES, LANES)] * ah)
            return 0
        lax.fori_loop(0, W // 2, ebody, 0)
        pltpu.sync_copy(hg, out_t.at[idb.at[1]], add=True)

    b_pref(0, 0)

    def b_steady(g, _):
        b_step(2 * g, 0)
        b_step(2 * g + 1, 1)
        return 0
    lax.fori_loop(0, NWIN // 2, b_steady, 0)
    b_wait_pref(0)
    plsc.subcore_barrier()

    def wb(k, _):
        pltpu.sync_copy(out_t.at[pl.ds(r0 + k * W, W)], hg)
        pltpu.sync_copy(hg, seg_hbm.at[c, pl.ds(r0 + k * W, W)])
        return 0
    lax.fori_loop(0, RPT // W, wb, 0)


@functools.partial(jax.jit, static_argnames=())
def _sc_layer(hp2, als16, ald16, sdx, z16, z64):
    mesh = plsc.VectorSubcoreMesh(core_axis_name="c", subcore_axis_name="s",
                                  num_cores=NC, num_subcores=NS)
    seg, _ex = pl.kernel(
        _sc_body,
        out_type=[
            jax.ShapeDtypeStruct((NC, NPAD, 64), _f32),
            jax.ShapeDtypeStruct((NC, EPAD + W, 16), _f32),
        ],
        mesh=mesh,
        scratch_types=[
            pltpu.VMEM_SHARED((NPAD, 64), _f32),
            pltpu.VMEM_SHARED((NPAD, 64), _f32),
            pltpu.VMEM_SHARED((NPAD, 16), _f32),
            pltpu.VMEM_SHARED((NPAD, 16), _f32),
            pltpu.VMEM_SHARED((NPAD, 16), _f32),
            pltpu.VMEM((2, W), jnp.int32),
            pltpu.VMEM((2, W), jnp.int32),
            pltpu.VMEM((W, 16), _f32),
            pltpu.VMEM((W, 16), _f32),
            pltpu.VMEM((W, 16), _f32),
            pltpu.VMEM((W, 64), _f32),
            pltpu.SemaphoreType.DMA,
            pltpu.SemaphoreType.DMA,
            pltpu.SemaphoreType.DMA,
            pltpu.SemaphoreType.DMA,
        ],
        compiler_params=pltpu.CompilerParams(use_tc_tiling_on_sc=False),
    )(hp2, als16, ald16, sdx, z16, z64)
    return seg


def _mk_scat(att_s, att_d):
    eye = jnp.eye(H, dtype=_f32)
    ss = (att_s[:, :, None] * eye[:, None, :]).reshape(D, H)
    sd = (att_d[:, :, None] * eye[:, None, :]).reshape(D, H)
    return jnp.concatenate([ss, sd], axis=1)


def _split_tables(hp, al):
    hp2 = jnp.pad(hp.reshape(N, 2, 64).transpose(1, 0, 2),
                  ((0, 0), (0, NPAD - N), (0, 0)))
    als = jnp.pad(al[:, :H].reshape(N, 2, 4).transpose(1, 0, 2),
                  ((0, 0), (0, NPAD - N), (0, 12)))
    ald = jnp.pad(al[:, H:].reshape(N, 2, 4).transpose(1, 0, 2),
                  ((0, 0), (0, NPAD - N), (0, 12)))
    return hp2, als, ald


def kernel(x, edge_attr, w0, w_rest, att_src, att_dst, bias, gamma, beta,
           edge_index, batch):
    loops = jnp.arange(N, dtype=jnp.int32)
    padi = jnp.arange(PADE, dtype=jnp.int32)
    zw = jnp.zeros((W,), jnp.int32)
    src = jnp.concatenate([edge_index[0], loops, padi % N, zw])
    dst = jnp.concatenate([edge_index[1], loops, N + padi % PAD_ROWS, zw])
    sdx = jnp.stack([src.reshape(-1, W), dst.reshape(-1, W)], axis=1)

    z16 = jnp.zeros((NPAD, 16), _f32)
    z64 = jnp.zeros((NPAD, 64), _f32)
    batch3 = batch.reshape(N // _RB, 1, _RB)

    hp, al = _xform1(x, w0, _mk_scat(att_src[0], att_dst[0]))
    hp2, als16, ald16 = _split_tables(hp, al)
    seg = _sc_layer(hp2, als16, ald16, sdx, z16, z64)
    h, stats = _combine(seg, jnp.zeros((N, D), _f32), bias[0:1])

    for l in range(1, L):
        gb = jnp.stack([gamma[l - 1], beta[l - 1]])
        hp, al = _xform(h, stats, gb, w_rest[l - 1],
                        _mk_scat(att_src[l], att_dst[l]))
        hp2, als16, ald16 = _split_tables(hp, al)
        seg = _sc_layer(hp2, als16, ald16, sdx, z16, z64)
        h, stats = _combine(seg, h, bias[l:l + 1])

    gb = jnp.stack([gamma[L - 1], beta[L - 1]])
    return _pool(h, stats, gb, batch3)


# speedup vs baseline: 67.6962x; 1.0277x over previous
import functools

import jax
import jax.numpy as jnp
from jax import lax
from jax.experimental import pallas as pl
from jax.experimental.pallas import tpu as pltpu
from jax.experimental.pallas import tpu_sc as plsc

N = 10000
E = 320000
H = 8
C = 16
D = 128
L = 5
G = 256

NC = 2
NS = 16
LANES = 16

NPAD = 10240
W = 128
NWIN = 162
ET = NWIN * W
EPAD = ET * NS
PADE = EPAD - E - N
PAD_ROWS = NPAD - N

RPT = NPAD // NS

_f32 = jnp.float32


def _xform_body(h_ref, stats_ref, gb_ref, w_ref, s_ref, hp_ref, al_ref):
    h = h_ref[...]
    st = stats_ref[...]
    mu = st[0:1] / N
    var = st[1:2] / N - mu * mu
    act = gb_ref[0:1] * (h - mu) * lax.rsqrt(var + 1e-5) + gb_ref[1:2]
    act = jnp.maximum(act, 0.0)
    hp = jnp.dot(act, w_ref[...], preferred_element_type=_f32)
    hp_ref[...] = hp
    al_ref[...] = jnp.dot(hp, s_ref[...], preferred_element_type=_f32)


def _xform1_body(x_ref, w_ref, s_ref, hp_ref, al_ref):
    hp = jnp.dot(x_ref[...], w_ref[...], preferred_element_type=_f32)
    hp_ref[...] = hp
    al_ref[...] = jnp.dot(hp, s_ref[...], preferred_element_type=_f32)


def _combine_body(seg_ref, hprev_ref, bias_ref, h_ref, stats_ref, acc):
    i = pl.program_id(0)
    h = (jnp.concatenate([seg_ref[0], seg_ref[1]], axis=1)
         + bias_ref[...] + hprev_ref[...])
    h_ref[...] = h

    @pl.when(i == 0)
    def _():
        acc[...] = jnp.zeros_like(acc)

    acc[0:1, :] += jnp.sum(h, axis=0, keepdims=True)
    acc[1:2, :] += jnp.sum(h * h, axis=0, keepdims=True)

    @pl.when(i == pl.num_programs(0) - 1)
    def _():
        stats_ref[...] = acc[...]


def _pool_body(h_ref, stats_ref, gb_ref, batch_ref, out_ref):
    i = pl.program_id(0)
    st = stats_ref[...]
    mu = st[0:1] / N
    var = st[1:2] / N - mu * mu
    hb = gb_ref[0:1] * (h_ref[...] - mu) * lax.rsqrt(var + 1e-5) + gb_ref[1:2]
    bt = batch_ref[0]
    gid = lax.broadcasted_iota(jnp.int32, (G, bt.shape[1]), 0)
    oh = (bt == gid).astype(_f32)
    contrib = lax.dot_general(oh, hb, (((1,), (0,)), ((), ())),
                              preferred_element_type=_f32)

    @pl.when(i == 0)
    def _():
        out_ref[...] = jnp.zeros_like(out_ref)

    out_ref[...] += contrib


_RB = 1000


def _xform(h, stats, gb, w, scat):
    nb = N // _RB
    return pl.pallas_call(
        _xform_body,
        grid=(nb,),
        in_specs=[
            pl.BlockSpec((_RB, D), lambda i: (i, 0)),
            pl.BlockSpec((2, D), lambda i: (0, 0)),
            pl.BlockSpec((2, D), lambda i: (0, 0)),
            pl.BlockSpec((D, D), lambda i: (0, 0)),
            pl.BlockSpec((D, 2 * H), lambda i: (0, 0)),
        ],
        out_specs=[
            pl.BlockSpec((_RB, D), lambda i: (i, 0)),
            pl.BlockSpec((_RB, 2 * H), lambda i: (i, 0)),
        ],
        out_shape=[
            jax.ShapeDtypeStruct((N, D), _f32),
            jax.ShapeDtypeStruct((N, 2 * H), _f32),
        ],
    )(h, stats, gb, w, scat)


def _xform1(x, w, scat):
    nb = N // _RB
    return pl.pallas_call(
        _xform1_body,
        grid=(nb,),
        in_specs=[
            pl.BlockSpec((_RB, 8), lambda i: (i, 0)),
            pl.BlockSpec((8, D), lambda i: (0, 0)),
            pl.BlockSpec((D, 2 * H), lambda i: (0, 0)),
        ],
        out_specs=[
            pl.BlockSpec((_RB, D), lambda i: (i, 0)),
            pl.BlockSpec((_RB, 2 * H), lambda i: (i, 0)),
        ],
        out_shape=[
            jax.ShapeDtypeStruct((N, D), _f32),
            jax.ShapeDtypeStruct((N, 2 * H), _f32),
        ],
    )(x, w, scat)


def _combine(seg, hprev, bias):
    nb = N // _RB
    return pl.pallas_call(
        _combine_body,
        grid=(nb,),
        in_specs=[
            pl.BlockSpec((2, _RB, 64), lambda i: (0, i, 0)),
            pl.BlockSpec((_RB, D), lambda i: (i, 0)),
            pl.BlockSpec((1, D), lambda i: (0, 0)),
        ],
        out_specs=[
            pl.BlockSpec((_RB, D), lambda i: (i, 0)),
            pl.BlockSpec((2, D), lambda i: (0, 0)),
        ],
        out_shape=[
            jax.ShapeDtypeStruct((N, D), _f32),
            jax.ShapeDtypeStruct((2, D), _f32),
        ],
        scratch_shapes=[pltpu.VMEM((2, D), _f32)],
    )(seg, hprev, bias)


def _pool(h, stats, gb, batch3):
    nb = N // _RB
    return pl.pallas_call(
        _pool_body,
        grid=(nb,),
        in_specs=[
            pl.BlockSpec((_RB, D), lambda i: (i, 0)),
            pl.BlockSpec((2, D), lambda i: (0, 0)),
            pl.BlockSpec((2, D), lambda i: (0, 0)),
            pl.BlockSpec((1, 1, _RB), lambda i: (i, 0, 0)),
        ],
        out_specs=pl.BlockSpec((G, D), lambda i: (0, 0)),
        out_shape=jax.ShapeDtypeStruct((G, D), _f32),
    )(h, stats, gb, batch3)


def _bcast_lane(v, h):
    idx = lax.full((LANES, 1), h, jnp.int32)
    dn = lax.GatherDimensionNumbers(
        offset_dims=(), collapsed_slice_dims=(0,), start_index_map=(0,))
    return lax.gather(v, idx, dn, (1,),
                      mode=lax.GatherScatterMode.PROMISE_IN_BOUNDS)


def _sc_body(hp_hbm, als_hbm, ald_hbm, sdx_hbm, z16_hbm, z64_hbm,
             seg_hbm, exh_hbm,
             hp_t, out_t, als_t, ald_t, den_t,
             id0, id1, rs0, rs1, rd, hg, sem_a, sem_b, sp0, sp1):
    ids = (id0, id1)
    rss = (rs0, rs1)
    sps = (sp0, sp1)
    rs = rs0
    c = lax.axis_index("c")
    s = lax.axis_index("s")
    r0 = s * RPT
    tbase = s * ET

    def stage16(hview, tview):
        def body(k, _):
            pltpu.sync_copy(hview.at[pl.ds(r0 + k * W, W)], rs)
            pltpu.sync_copy(rs, tview.at[pl.ds(r0 + k * W, W)])
            return 0
        lax.fori_loop(0, RPT // W, body, 0)

    def stage64(hview, tview):
        def body(k, _):
            pltpu.sync_copy(hview.at[pl.ds(r0 + k * W, W)], hg)
            pltpu.sync_copy(hg, tview.at[pl.ds(r0 + k * W, W)])
            return 0
        lax.fori_loop(0, RPT // W, body, 0)

    stage64(hp_hbm.at[c], hp_t)
    stage16(als_hbm.at[c], als_t)
    stage16(ald_hbm.at[c], ald_t)
    stage16(z16_hbm, den_t)
    stage64(z64_hbm, out_t)
    plsc.subcore_barrier()

    wbase = s * NWIN

    def a_pref(w, b):
        pltpu.async_copy(sdx_hbm.at[wbase + w], ids[b], sps[b])

    def a_wait_pref(b):
        pltpu.make_async_copy(sdx_hbm.at[0], ids[b], sps[b]).wait()

    def a_step(w, b):
        a_pref(w + 1, 1 - b)
        a_wait_pref(b)
        idb = ids[b]
        rsb = rss[b]
        ca = pltpu.async_copy(als_t.at[idb.at[0]], rsb, sem_a)
        cb = pltpu.async_copy(ald_t.at[idb.at[1]], rd, sem_b)
        ca.wait()
        cb.wait()

        def ebody(j, _):
            for u in range(8):
                e = j * 8 + u
                v = rsb[e] + rd[e]
                v = jnp.maximum(v, 0.2 * v)
                rsb[e] = jnp.exp(v)
            return 0
        lax.fori_loop(0, W // 8, ebody, 0)
        eb = tbase + w * W
        cs = pltpu.async_copy(rsb, den_t.at[idb.at[1]], sem_a, add=True)
        ce = pltpu.async_copy(rsb, exh_hbm.at[c, pl.ds(eb, W)], sem_b)
        cs.wait()
        ce.wait()

    a_pref(0, 0)

    def a_steady(g, _):
        a_step(2 * g, 0)
        a_step(2 * g + 1, 1)
        return 0
    lax.fori_loop(0, NWIN // 2, a_steady, 0)
    a_wait_pref(0)
    plsc.subcore_barrier()

    def dchunk(k, _):
        pltpu.sync_copy(den_t.at[pl.ds(r0 + k * W, W)], rs)

        def dbody(j, _):
            rs[j] = 1.0 / (rs[j] + 1e-16)
            return 0
        lax.fori_loop(0, W, dbody, 0)
        pltpu.sync_copy(rs, den_t.at[pl.ds(r0 + k * W, W)])
        return 0
    lax.fori_loop(0, RPT // W, dchunk, 0)
    plsc.subcore_barrier()

    def b_pref(w, b):
        eb = tbase + w * W
        pltpu.async_copy(sdx_hbm.at[wbase + w], ids[b], sps[b])
        pltpu.async_copy(exh_hbm.at[c, pl.ds(eb, W)], rss[b], sps[b])

    def b_wait_pref(b):
        pltpu.make_async_copy(sdx_hbm.at[0], ids[b], sps[b]).wait()
        pltpu.make_async_copy(exh_hbm.at[c, pl.ds(0, W)], rss[b],
                              sps[b]).wait()

    def b_step(w, b):
        b_pref(w + 1, 1 - b)
        b_wait_pref(b)
        idb = ids[b]
        rsb = rss[b]
        ca = pltpu.async_copy(den_t.at[idb.at[1]], rd, sem_a)
        cb = pltpu.async_copy(hp_t.at[idb.at[0]], hg, sem_b)
        ca.wait()
        cb.wait()

        def ebody(j, _):
            for u in range(2):
                e = j * 2 + u
                alpha = rsb[e] * rd[e]
                for h in range(4):
                    ah = _bcast_lane(alpha, h)
                    hg[e, pl.ds(h * LANES, LANES)] = (
                        hg[e, pl.ds(h * LANES, LANES)] * ah)
            return 0
        lax.fori_loop(0, W // 2, ebody, 0)
        pltpu.sync_copy(hg, out_t.at[idb.at[1]], add=True)

    b_pref(0, 0)

    def b_steady(g, _):
        b_step(2 * g, 0)
        b_step(2 * g + 1, 1)
        return 0
    lax.fori_loop(0, NWIN // 2, b_steady, 0)
    b_wait_pref(0)
    plsc.subcore_barrier()

    def wb(k, _):
        pltpu.sync_copy(out_t.at[pl.ds(r0 + k * W, W)], hg)
        pltpu.sync_copy(hg, seg_hbm.at[c, pl.ds(r0 + k * W, W)])
        return 0
    lax.fori_loop(0, RPT // W, wb, 0)


@functools.partial(jax.jit, static_argnames=())
def _sc_layer(hp2, als16, ald16, sdx, z16, z64):
    mesh = plsc.VectorSubcoreMesh(core_axis_name="c", subcore_axis_name="s",
                                  num_cores=NC, num_subcores=NS)
    seg, _ex = pl.kernel(
        _sc_body,
        out_type=[
            jax.ShapeDtypeStruct((NC, NPAD, 64), _f32),
            jax.ShapeDtypeStruct((NC, EPAD + W, 16), _f32),
        ],
        mesh=mesh,
        scratch_types=[
            pltpu.VMEM_SHARED((NPAD, 64), _f32),
            pltpu.VMEM_SHARED((NPAD, 64), _f32),
            pltpu.VMEM_SHARED((NPAD, 16), _f32),
            pltpu.VMEM_SHARED((NPAD, 16), _f32),
            pltpu.VMEM_SHARED((NPAD, 16), _f32),
            pltpu.VMEM((2, W), jnp.int32),
            pltpu.VMEM((2, W), jnp.int32),
            pltpu.VMEM((W, 16), _f32),
            pltpu.VMEM((W, 16), _f32),
            pltpu.VMEM((W, 16), _f32),
            pltpu.VMEM((W, 64), _f32),
            pltpu.SemaphoreType.DMA,
            pltpu.SemaphoreType.DMA,
            pltpu.SemaphoreType.DMA,
            pltpu.SemaphoreType.DMA,
        ],
        compiler_params=pltpu.CompilerParams(use_tc_tiling_on_sc=False),
    )(hp2, als16, ald16, sdx, z16, z64)
    return seg


def _mk_scat(att_s, att_d):
    eye = jnp.eye(H, dtype=_f32)
    ss = (att_s[:, :, None] * eye[:, None, :]).reshape(D, H)
    sd = (att_d[:, :, None] * eye[:, None, :]).reshape(D, H)
    return jnp.concatenate([ss, sd], axis=1)


def _split_tables(hp, al):
    hp2 = jnp.pad(hp.reshape(N, 2, 64).transpose(1, 0, 2),
                  ((0, 0), (0, NPAD - N), (0, 0)))
    als = jnp.pad(al[:, :H].reshape(N, 2, 4).transpose(1, 0, 2),
                  ((0, 0), (0, NPAD - N), (0, 12)))
    ald = jnp.pad(al[:, H:].reshape(N, 2, 4).transpose(1, 0, 2),
                  ((0, 0), (0, NPAD - N), (0, 12)))
    return hp2, als, ald


def kernel(x, edge_attr, w0, w_rest, att_src, att_dst, bias, gamma, beta,
           edge_index, batch):
    loops = jnp.arange(N, dtype=jnp.int32)
    padi = jnp.arange(PADE, dtype=jnp.int32)
    zw = jnp.zeros((W,), jnp.int32)
    src = jnp.concatenate([edge_index[0], loops, padi % N, zw])
    dst = jnp.concatenate([edge_index[1], loops, N + padi % PAD_ROWS, zw])
    sdx = jnp.stack([src.reshape(-1, W), dst.reshape(-1, W)], axis=1)

    z16 = jnp.zeros((NPAD, 16), _f32)
    z64 = jnp.zeros((NPAD, 64), _f32)
    batch3 = batch.reshape(N // _RB, 1, _RB)

    hp, al = _xform1(x, w0, _mk_scat(att_src[0], att_dst[0]))
    hp2, als16, ald16 = _split_tables(hp, al)
    seg = _sc_layer(hp2, als16, ald16, sdx, z16, z64)
    h, stats = _combine(seg, jnp.zeros((N, D), _f32), bias[0:1])

    for l in range(1, L):
        gb = jnp.stack([gamma[l - 1], beta[l - 1]])
        hp, al = _xform(h, stats, gb, w_rest[l - 1],
                        _mk_scat(att_src[l], att_dst[l]))
        hp2, als16, ald16 = _split_tables(hp, al)
        seg = _sc_layer(hp2, als16, ald16, sdx, z16, z64)
        h, stats = _combine(seg, h, bias[l:l + 1])

    gb = jnp.stack([gamma[L - 1], beta[L - 1]])
    return _pool(h, stats, gb, batch3)


# speedup vs baseline: 68.2109x; 1.0076x over previous
import functools

import jax
import jax.numpy as jnp
from jax import lax
from jax.experimental import pallas as pl
from jax.experimental.pallas import tpu as pltpu
from jax.experimental.pallas import tpu_sc as plsc

N = 10000
E = 320000
H = 8
C = 16
D = 128
L = 5
G = 256

NC = 2
NS = 16
LANES = 16

NPAD = 10240
W = 128
NWIN = 162
ET = NWIN * W
EPAD = ET * NS
PADE = EPAD - E - N
PAD_ROWS = NPAD - N

RPT = NPAD // NS

_f32 = jnp.float32


def _xform_body(h_ref, stats_ref, gb_ref, w_ref, s_ref, hp_ref, al_ref):
    h = h_ref[...]
    st = stats_ref[...]
    mu = st[0:1] / N
    var = st[1:2] / N - mu * mu
    act = gb_ref[0:1] * (h - mu) * lax.rsqrt(var + 1e-5) + gb_ref[1:2]
    act = jnp.maximum(act, 0.0)
    hp = jnp.dot(act, w_ref[...], preferred_element_type=_f32)
    hp_ref[...] = hp
    al_ref[...] = jnp.dot(hp, s_ref[...], preferred_element_type=_f32)


def _xform1_body(x_ref, w_ref, s_ref, hp_ref, al_ref):
    hp = jnp.dot(x_ref[...], w_ref[...], preferred_element_type=_f32)
    hp_ref[...] = hp
    al_ref[...] = jnp.dot(hp, s_ref[...], preferred_element_type=_f32)


def _combine_body(seg_ref, hprev_ref, bias_ref, h_ref, stats_ref, acc):
    i = pl.program_id(0)
    h = (jnp.concatenate([seg_ref[0], seg_ref[1]], axis=1)
         + bias_ref[...] + hprev_ref[...])
    h_ref[...] = h

    @pl.when(i == 0)
    def _():
        acc[...] = jnp.zeros_like(acc)

    acc[0:1, :] += jnp.sum(h, axis=0, keepdims=True)
    acc[1:2, :] += jnp.sum(h * h, axis=0, keepdims=True)

    @pl.when(i == pl.num_programs(0) - 1)
    def _():
        stats_ref[...] = acc[...]


def _pool_body(h_ref, stats_ref, gb_ref, batch_ref, out_ref):
    i = pl.program_id(0)
    st = stats_ref[...]
    mu = st[0:1] / N
    var = st[1:2] / N - mu * mu
    hb = gb_ref[0:1] * (h_ref[...] - mu) * lax.rsqrt(var + 1e-5) + gb_ref[1:2]
    bt = batch_ref[0]
    gid = lax.broadcasted_iota(jnp.int32, (G, bt.shape[1]), 0)
    oh = (bt == gid).astype(_f32)
    contrib = lax.dot_general(oh, hb, (((1,), (0,)), ((), ())),
                              preferred_element_type=_f32)

    @pl.when(i == 0)
    def _():
        out_ref[...] = jnp.zeros_like(out_ref)

    out_ref[...] += contrib


_RB = 1000


def _xform(h, stats, gb, w, scat):
    nb = N // _RB
    return pl.pallas_call(
        _xform_body,
        grid=(nb,),
        in_specs=[
            pl.BlockSpec((_RB, D), lambda i: (i, 0)),
            pl.BlockSpec((2, D), lambda i: (0, 0)),
            pl.BlockSpec((2, D), lambda i: (0, 0)),
            pl.BlockSpec((D, D), lambda i: (0, 0)),
            pl.BlockSpec((D, 2 * H), lambda i: (0, 0)),
        ],
        out_specs=[
            pl.BlockSpec((_RB, D), lambda i: (i, 0)),
            pl.BlockSpec((_RB, 2 * H), lambda i: (i, 0)),
        ],
        out_shape=[
            jax.ShapeDtypeStruct((N, D), _f32),
            jax.ShapeDtypeStruct((N, 2 * H), _f32),
        ],
    )(h, stats, gb, w, scat)


def _xform1(x, w, scat):
    nb = N // _RB
    return pl.pallas_call(
        _xform1_body,
        grid=(nb,),
        in_specs=[
            pl.BlockSpec((_RB, 8), lambda i: (i, 0)),
            pl.BlockSpec((8, D), lambda i: (0, 0)),
            pl.BlockSpec((D, 2 * H), lambda i: (0, 0)),
        ],
        out_specs=[
            pl.BlockSpec((_RB, D), lambda i: (i, 0)),
            pl.BlockSpec((_RB, 2 * H), lambda i: (i, 0)),
        ],
        out_shape=[
            jax.ShapeDtypeStruct((N, D), _f32),
            jax.ShapeDtypeStruct((N, 2 * H), _f32),
        ],
    )(x, w, scat)


def _combine(seg, hprev, bias):
    nb = N // _RB
    return pl.pallas_call(
        _combine_body,
        grid=(nb,),
        in_specs=[
            pl.BlockSpec((2, _RB, 64), lambda i: (0, i, 0)),
            pl.BlockSpec((_RB, D), lambda i: (i, 0)),
            pl.BlockSpec((1, D), lambda i: (0, 0)),
        ],
        out_specs=[
            pl.BlockSpec((_RB, D), lambda i: (i, 0)),
            pl.BlockSpec((2, D), lambda i: (0, 0)),
        ],
        out_shape=[
            jax.ShapeDtypeStruct((N, D), _f32),
            jax.ShapeDtypeStruct((2, D), _f32),
        ],
        scratch_shapes=[pltpu.VMEM((2, D), _f32)],
    )(seg, hprev, bias)


def _pool(h, stats, gb, batch3):
    nb = N // _RB
    return pl.pallas_call(
        _pool_body,
        grid=(nb,),
        in_specs=[
            pl.BlockSpec((_RB, D), lambda i: (i, 0)),
            pl.BlockSpec((2, D), lambda i: (0, 0)),
            pl.BlockSpec((2, D), lambda i: (0, 0)),
            pl.BlockSpec((1, 1, _RB), lambda i: (i, 0, 0)),
        ],
        out_specs=pl.BlockSpec((G, D), lambda i: (0, 0)),
        out_shape=jax.ShapeDtypeStruct((G, D), _f32),
    )(h, stats, gb, batch3)


def _bcast_lane(v, h):
    idx = lax.full((LANES, 1), h, jnp.int32)
    dn = lax.GatherDimensionNumbers(
        offset_dims=(), collapsed_slice_dims=(0,), start_index_map=(0,))
    return lax.gather(v, idx, dn, (1,),
                      mode=lax.GatherScatterMode.PROMISE_IN_BOUNDS)


def _sc_body(hp_hbm, als_hbm, ald_hbm, sdx_hbm, z16_hbm, z64_hbm,
             seg_hbm, exh_hbm,
             hp_t, out_t, als_t, ald_t, den_t,
             id0, id1, rs0, rs1, rd, hg, sem_a, sem_b, sp0, sp1):
    ids = (id0, id1)
    rss = (rs0, rs1)
    sps = (sp0, sp1)
    rs = rs0
    c = lax.axis_index("c")
    s = lax.axis_index("s")
    r0 = s * RPT
    tbase = s * ET

    def stage16(hview, tview):
        def body(k, _):
            pltpu.sync_copy(hview.at[pl.ds(r0 + k * W, W)], rs)
            pltpu.sync_copy(rs, tview.at[pl.ds(r0 + k * W, W)])
            return 0
        lax.fori_loop(0, RPT // W, body, 0)

    def stage64(hview, tview):
        def body(k, _):
            pltpu.sync_copy(hview.at[pl.ds(r0 + k * W, W)], hg)
            pltpu.sync_copy(hg, tview.at[pl.ds(r0 + k * W, W)])
            return 0
        lax.fori_loop(0, RPT // W, body, 0)

    stage64(hp_hbm.at[c], hp_t)
    stage16(als_hbm.at[c], als_t)
    stage16(ald_hbm.at[c], ald_t)
    stage16(z16_hbm, den_t)
    stage64(z64_hbm, out_t)
    plsc.subcore_barrier()

    wbase = s * NWIN

    def a_pref(w, b):
        pltpu.async_copy(sdx_hbm.at[wbase + w], ids[b], sps[b])

    def a_wait_pref(b):
        pltpu.make_async_copy(sdx_hbm.at[0], ids[b], sps[b]).wait()

    def a_step(w, b):
        a_pref(w + 1, 1 - b)
        a_wait_pref(b)
        idb = ids[b]
        rsb = rss[b]
        ca = pltpu.async_copy(als_t.at[idb.at[0]], rsb, sem_a)
        cb = pltpu.async_copy(ald_t.at[idb.at[1]], rd, sem_b)
        ca.wait()
        cb.wait()

        def ebody(j, _):
            for u in range(8):
                e = j * 8 + u
                v = rsb[e] + rd[e]
                v = jnp.maximum(v, 0.2 * v)
                rsb[e] = jnp.exp(v)
            return 0
        lax.fori_loop(0, W // 8, ebody, 0)
        eb = tbase + w * W
        cs = pltpu.async_copy(rsb, den_t.at[idb.at[1]], sem_a, add=True)
        ce = pltpu.async_copy(rsb, exh_hbm.at[c, pl.ds(eb, W)], sem_b)
        cs.wait()
        ce.wait()

    a_pref(0, 0)

    def a_steady(g, _):
        a_step(2 * g, 0)
        a_step(2 * g + 1, 1)
        return 0
    lax.fori_loop(0, NWIN // 2, a_steady, 0)
    a_wait_pref(0)
    plsc.subcore_barrier()

    def dchunk(k, _):
        pltpu.sync_copy(den_t.at[pl.ds(r0 + k * W, W)], rs)

        def dbody(j, _):
            rs[j] = 1.0 / (rs[j] + 1e-16)
            return 0
        lax.fori_loop(0, W, dbody, 0)
        pltpu.sync_copy(rs, den_t.at[pl.ds(r0 + k * W, W)])
        return 0
    lax.fori_loop(0, RPT // W, dchunk, 0)
    plsc.subcore_barrier()

    def b_pref(w, b):
        eb = tbase + w * W
        pltpu.async_copy(sdx_hbm.at[wbase + w], ids[b], sps[b])
        pltpu.async_copy(exh_hbm.at[c, pl.ds(eb, W)], rss[b], sps[b])

    def b_wait_pref(b):
        pltpu.make_async_copy(sdx_hbm.at[0], ids[b], sps[b]).wait()
        pltpu.make_async_copy(exh_hbm.at[c, pl.ds(0, W)], rss[b],
                              sps[b]).wait()

    def b_step(w, b):
        b_pref(w + 1, 1 - b)
        b_wait_pref(b)
        idb = ids[b]
        rsb = rss[b]
        ca = pltpu.async_copy(den_t.at[idb.at[1]], rd, sem_a)
        cb = pltpu.async_copy(hp_t.at[idb.at[0]], hg, sem_b)
        ca.wait()
        cb.wait()

        def ebody(j, _):
            for u in range(4):
                e = j * 4 + u
                alpha = rsb[e] * rd[e]
                for h in range(4):
                    ah = _bcast_lane(alpha, h)
                    hg[e, pl.ds(h * LANES, LANES)] = (
                        hg[e, pl.ds(h * LANES, LANES)] * ah)
            return 0
        lax.fori_loop(0, W // 4, ebody, 0)
        pltpu.sync_copy(hg, out_t.at[idb.at[1]], add=True)

    b_pref(0, 0)

    def b_steady(g, _):
        b_step(2 * g, 0)
        b_step(2 * g + 1, 1)
        return 0
    lax.fori_loop(0, NWIN // 2, b_steady, 0)
    b_wait_pref(0)
    plsc.subcore_barrier()

    def wb(k, _):
        pltpu.sync_copy(out_t.at[pl.ds(r0 + k * W, W)], hg)
        pltpu.sync_copy(hg, seg_hbm.at[c, pl.ds(r0 + k * W, W)])
        return 0
    lax.fori_loop(0, RPT // W, wb, 0)


@functools.partial(jax.jit, static_argnames=())
def _sc_layer(hp2, als16, ald16, sdx, z16, z64):
    mesh = plsc.VectorSubcoreMesh(core_axis_name="c", subcore_axis_name="s",
                                  num_cores=NC, num_subcores=NS)
    seg, _ex = pl.kernel(
        _sc_body,
        out_type=[
            jax.ShapeDtypeStruct((NC, NPAD, 64), _f32),
            jax.ShapeDtypeStruct((NC, EPAD + W, 16), _f32),
        ],
        mesh=mesh,
        scratch_types=[
            pltpu.VMEM_SHARED((NPAD, 64), _f32),
            pltpu.VMEM_SHARED((NPAD, 64), _f32),
            pltpu.VMEM_SHARED((NPAD, 16), _f32),
            pltpu.VMEM_SHARED((NPAD, 16), _f32),
            pltpu.VMEM_SHARED((NPAD, 16), _f32),
            pltpu.VMEM((2, W), jnp.int32),
            pltpu.VMEM((2, W), jnp.int32),
            pltpu.VMEM((W, 16), _f32),
            pltpu.VMEM((W, 16), _f32),
            pltpu.VMEM((W, 16), _f32),
            pltpu.VMEM((W, 64), _f32),
            pltpu.SemaphoreType.DMA,
            pltpu.SemaphoreType.DMA,
            pltpu.SemaphoreType.DMA,
            pltpu.SemaphoreType.DMA,
        ],
        compiler_params=pltpu.CompilerParams(use_tc_tiling_on_sc=False),
    )(hp2, als16, ald16, sdx, z16, z64)
    return seg


def _mk_scat(att_s, att_d):
    eye = jnp.eye(H, dtype=_f32)
    ss = (att_s[:, :, None] * eye[:, None, :]).reshape(D, H)
    sd = (att_d[:, :, None] * eye[:, None, :]).reshape(D, H)
    return jnp.concatenate([ss, sd], axis=1)


def _split_tables(hp, al):
    hp2 = jnp.pad(hp.reshape(N, 2, 64).transpose(1, 0, 2),
                  ((0, 0), (0, NPAD - N), (0, 0)))
    als = jnp.pad(al[:, :H].reshape(N, 2, 4).transpose(1, 0, 2),
                  ((0, 0), (0, NPAD - N), (0, 12)))
    ald = jnp.pad(al[:, H:].reshape(N, 2, 4).transpose(1, 0, 2),
                  ((0, 0), (0, NPAD - N), (0, 12)))
    return hp2, als, ald


def kernel(x, edge_attr, w0, w_rest, att_src, att_dst, bias, gamma, beta,
           edge_index, batch):
    loops = jnp.arange(N, dtype=jnp.int32)
    padi = jnp.arange(PADE, dtype=jnp.int32)
    zw = jnp.zeros((W,), jnp.int32)
    src = jnp.concatenate([edge_index[0], loops, padi % N, zw])
    dst = jnp.concatenate([edge_index[1], loops, N + padi % PAD_ROWS, zw])
    sdx = jnp.stack([src.reshape(-1, W), dst.reshape(-1, W)], axis=1)

    z16 = jnp.zeros((NPAD, 16), _f32)
    z64 = jnp.zeros((NPAD, 64), _f32)
    batch3 = batch.reshape(N // _RB, 1, _RB)

    hp, al = _xform1(x, w0, _mk_scat(att_src[0], att_dst[0]))
    hp2, als16, ald16 = _split_tables(hp, al)
    seg = _sc_layer(hp2, als16, ald16, sdx, z16, z64)
    h, stats = _combine(seg, jnp.zeros((N, D), _f32), bias[0:1])

    for l in range(1, L):
        gb = jnp.stack([gamma[l - 1], beta[l - 1]])
        hp, al = _xform(h, stats, gb, w_rest[l - 1],
                        _mk_scat(att_src[l], att_dst[l]))
        hp2, als16, ald16 = _split_tables(hp, al)
        seg = _sc_layer(hp2, als16, ald16, sdx, z16, z64)
        h, stats = _combine(seg, h, bias[l:l + 1])

    gb = jnp.stack([gamma[L - 1], beta[L - 1]])
    return _pool(h, stats, gb, batch3)
